# baseline (speedup 1.0000x reference)
import functools

import jax
import jax.numpy as jnp
from jax import lax
from jax.experimental import pallas as pl
from jax.experimental.pallas import tpu as pltpu
from jax.experimental.pallas import tpu_sc as plsc

NN = 4


def _topk_body(num_steps, tgt_ref, seq_ref, idx_out_ref, vals_ref, gidx_ref):
    step = pl.program_id(0)
    sb, b, d = seq_ref.shape

    @pl.when(step == 0)
    def _init():
        vals_ref[...] = jnp.full((NN, b), -jnp.inf, jnp.float32)
        gidx_ref[...] = jnp.zeros((NN, b), jnp.int32)

    seq = seq_ref[...]
    tgt = tgt_ref[...]
    scores = jnp.sum(seq * tgt[None], axis=-1)
    rowid = step * sb + lax.broadcasted_iota(jnp.int32, (sb, b), 0)

    x = jnp.concatenate([vals_ref[...], scores], axis=0)
    xi = jnp.concatenate([gidx_ref[...], rowid], axis=0)

    new_vals = []
    new_idx = []
    big = jnp.int32(2**30)
    for _ in range(NN):
        m = jnp.max(x, axis=0)
        sel = jnp.min(jnp.where(x == m[None], xi, big), axis=0)
        x = jnp.where(xi == sel[None], -jnp.inf, x)
        new_vals.append(m)
        new_idx.append(sel)
    vals_ref[...] = jnp.stack(new_vals, axis=0)
    gidx_ref[...] = jnp.stack(new_idx, axis=0)

    @pl.when(step == num_steps - 1)
    def _fin():
        col = lax.broadcasted_iota(jnp.int32, (NN, b), 1)
        idx_out_ref[...] = gidx_ref[...] * b + col


def _topk_indices(target_vec, seq_vecs, block_s=64):
    S, B, D = seq_vecs.shape
    num_steps = S // block_s
    return pl.pallas_call(
        functools.partial(_topk_body, num_steps),
        grid=(num_steps,),
        in_specs=[
            pl.BlockSpec((B, D), lambda i: (0, 0)),
            pl.BlockSpec((block_s, B, D), lambda i: (i, 0, 0)),
        ],
        out_specs=pl.BlockSpec((NN, B), lambda i: (0, 0)),
        out_shape=jax.ShapeDtypeStruct((NN, B), jnp.int32),
        scratch_shapes=[
            pltpu.VMEM((NN, B), jnp.float32),
            pltpu.VMEM((NN, B), jnp.int32),
        ],
    )(target_vec, seq_vecs)


def _sc_gather(table, flat_idx, n_rows, d):
    info = plsc.get_sparse_core_info()
    nw = info.num_cores * info.num_subcores
    per_w = n_rows // nw
    mesh = plsc.VectorSubcoreMesh(core_axis_name="c", subcore_axis_name="s")

    @functools.partial(
        pl.kernel,
        out_type=jax.ShapeDtypeStruct((n_rows, d), jnp.float32),
        mesh=mesh,
        scratch_types=[
            pltpu.VMEM((per_w,), jnp.int32),
            pltpu.VMEM((per_w, d), jnp.float32),
            pltpu.SemaphoreType.DMA,
        ],
    )
    def gather_kernel(table_hbm, idx_hbm, out_hbm, idx_v, rows_v, sem):
        wid = lax.axis_index("s") * info.num_cores + lax.axis_index("c")
        base = wid * per_w
        pltpu.sync_copy(idx_hbm.at[pl.ds(base, per_w)], idx_v)
        pltpu.async_copy(table_hbm.at[idx_v], rows_v, sem).wait()
        pltpu.sync_copy(rows_v, out_hbm.at[pl.ds(base, per_w)])

    return gather_kernel(table, flat_idx)


def kernel(target_vec, seq_vecs):
    S, B, D = seq_vecs.shape
    flat_idx = _topk_indices(target_vec, seq_vecs).reshape(-1)
    flat = seq_vecs.reshape(S * B, D)
    rows = _sc_gather(flat, flat_idx, NN * B, D)
    return rows.reshape(NN, B, D)
